# baseline (speedup 1.0000x reference)
import functools

import jax
import jax.numpy as jnp
import numpy as np
from jax import lax
from jax.experimental import pallas as pl
from jax.experimental.pallas import tpu as pltpu, tpu_sc as plsc

PRIME = (1 << 31) - 1
M31 = PRIME
B_ROWS = 1_000_000
W_SIZE = 125_000
BATCH = 16384
DIM = 32

_rng = np.random.RandomState(1234)
_HA0 = _rng.randint(1, PRIME, size=2)
_HC0 = _rng.randint(0, PRIME, size=2)
_HA1 = _rng.randint(1, PRIME, size=2)
_HC1 = _rng.randint(0, PRIME, size=2)

NW = 32
BPW = BATCH // NW
NH = 2
GCH = 128


def _rot31(t, s):
    if s == 0:
        return t
    return ((t << jnp.uint32(s)) & jnp.uint32(M31)) | (t >> jnp.uint32(31 - s))


def _red(u):
    return (u & jnp.uint32(M31)) + (u >> jnp.uint32(31))


def _hash16(xh, xl, a, c, m):
    ah = int(a) >> 16
    al = int(a) & 0xFFFF
    p1 = xh * jnp.uint32(ah)
    p2 = xh * jnp.uint32(al)
    p3 = xl * jnp.uint32(ah)
    p4 = xl * jnp.uint32(al)
    v = _rot31(p1, 26)
    v = _red(v + _rot31(p2, 10))
    v = _red(v + _rot31(p3, 16))
    v = _red(v + p4)
    v = _red(v + jnp.uint32(int(c)))
    v = jnp.where(v >= jnp.uint32(PRIME), v - jnp.uint32(PRIME), v)
    h = v.astype(jnp.int32)
    q = (h.astype(jnp.float32) * jnp.float32(1.0 / m)).astype(jnp.int32)
    r = h - q * jnp.int32(m)
    r = jnp.where(r < 0, r + jnp.int32(m), r)
    r = jnp.where(r >= jnp.int32(m), r - jnp.int32(m), r)
    return r


def _sc_body(x_hbm, table_hbm, w_hbm, out_hbm,
             x_v, it_v, iw_v, rows_v, wv_v, out_v, sem):
    wid = lax.axis_index("s") * 2 + lax.axis_index("c")
    base = wid * BPW

    pltpu.sync_copy(x_hbm.at[pl.ds(base, BPW)], x_v)

    def hash_body(j, _):
        off = j * 16
        xu = x_v[pl.ds(off, 16)].astype(jnp.uint32)
        xh = xu >> jnp.uint32(10)
        xl = xu & jnp.uint32(1023)
        it_v[pl.ds(off, 16)] = _hash16(xh, xl, _HA0[0], _HC0[0], B_ROWS)
        it_v[pl.ds(BPW + off, 16)] = _hash16(xh, xl, _HA0[1], _HC0[1], B_ROWS)
        iw_v[pl.ds(off, 16)] = _hash16(xh, xl, _HA1[0], _HC1[0], W_SIZE)
        iw_v[pl.ds(BPW + off, 16)] = _hash16(xh, xl, _HA1[1], _HC1[1], W_SIZE)
        return 0

    lax.fori_loop(jnp.int32(0), jnp.int32(BPW // 16), hash_body, 0,
                  unroll=False)

    cps = []
    for ch in range((NH * BPW) // GCH):
        cps.append(pltpu.async_copy(
            table_hbm.at[it_v.at[pl.ds(ch * GCH, GCH)]],
            rows_v.at[pl.ds(ch * GCH, GCH)], sem))
        cps.append(pltpu.async_copy(
            w_hbm.at[iw_v.at[pl.ds(ch * GCH, GCH)]],
            wv_v.at[pl.ds(ch * GCH, GCH)], sem))
    for cp in cps:
        cp.wait()

    def comb_body(b, _):
        bb = jnp.full((16,), 0, jnp.int32) + b
        w0 = plsc.load_gather(wv_v, [bb])
        w1 = plsc.load_gather(wv_v, [bb + BPW])
        r0a = rows_v[b, pl.ds(0, 16)]
        r0b = rows_v[b, pl.ds(16, 16)]
        r1a = rows_v[b + BPW, pl.ds(0, 16)]
        r1b = rows_v[b + BPW, pl.ds(16, 16)]
        out_v[b, pl.ds(0, 16)] = w0 * r0a + w1 * r1a
        out_v[b, pl.ds(16, 16)] = w0 * r0b + w1 * r1b
        return 0

    lax.fori_loop(jnp.int32(0), jnp.int32(BPW), comb_body, 0, unroll=False)

    pltpu.sync_copy(out_v, out_hbm.at[pl.ds(base, BPW)])


@jax.jit
def _run(x_i32, table, weights):
    mesh = plsc.VectorSubcoreMesh(core_axis_name="c", subcore_axis_name="s")
    f = functools.partial(
        pl.kernel,
        mesh=mesh,
        compiler_params=pltpu.CompilerParams(
            needs_layout_passes=False, use_tc_tiling_on_sc=False),
        out_type=jax.ShapeDtypeStruct((BATCH, DIM), jnp.float32),
        scratch_types=[
            pltpu.VMEM((BPW,), jnp.int32),
            pltpu.VMEM((NH * BPW,), jnp.int32),
            pltpu.VMEM((NH * BPW,), jnp.int32),
            pltpu.VMEM((NH * BPW, DIM), jnp.float32),
            pltpu.VMEM((NH * BPW,), jnp.float32),
            pltpu.VMEM((BPW, DIM), jnp.float32),
            pltpu.SemaphoreType.DMA,
        ],
    )(_sc_body)
    return f(x_i32, table, weights)


def kernel(x, table, weights):
    return _run(x.astype(jnp.int32), table, weights)


# speedup vs baseline: 2.1124x; 2.1124x over previous
import functools

import jax
import jax.numpy as jnp
import numpy as np
from jax import lax
from jax.experimental import pallas as pl
from jax.experimental.pallas import tpu as pltpu, tpu_sc as plsc

PRIME = (1 << 31) - 1
M31 = PRIME
B_ROWS = 1_000_000
W_SIZE = 125_000
BATCH = 16384
DIM = 32

_rng = np.random.RandomState(1234)
_HA0 = _rng.randint(1, PRIME, size=2)
_HC0 = _rng.randint(0, PRIME, size=2)
_HA1 = _rng.randint(1, PRIME, size=2)
_HC1 = _rng.randint(0, PRIME, size=2)

NW = 32
BPW = BATCH // NW
NH = 2
WIN = 16
NWIN = BPW // WIN


def _rot31(t, s):
    if s == 0:
        return t
    return ((t << jnp.uint32(s)) & jnp.uint32(M31)) | (t >> jnp.uint32(31 - s))


def _red(u):
    return (u & jnp.uint32(M31)) + (u >> jnp.uint32(31))


def _hash16(xh, xl, a, c, m):
    ah = int(a) >> 16
    al = int(a) & 0xFFFF
    p1 = xh * jnp.uint32(ah)
    p2 = xh * jnp.uint32(al)
    p3 = xl * jnp.uint32(ah)
    p4 = xl * jnp.uint32(al)
    v = _rot31(p1, 26)
    v = _red(v + _rot31(p2, 10))
    v = _red(v + _rot31(p3, 16))
    v = _red(v + p4)
    v = _red(v + jnp.uint32(int(c)))
    v = jnp.where(v >= jnp.uint32(PRIME), v - jnp.uint32(PRIME), v)
    h = v.astype(jnp.int32)
    q = (h.astype(jnp.float32) * jnp.float32(1.0 / m)).astype(jnp.int32)
    r = h - q * jnp.int32(m)
    r = jnp.where(r < 0, r + jnp.int32(m), r)
    r = jnp.where(r >= jnp.int32(m), r - jnp.int32(m), r)
    return r


def _sc_body(x_hbm, table_hbm, w_hbm, out_hbm,
             x_v, itile_v, iline_v, iw_v, wv_v, out_v,
             blkA0, blkA1, blkB0, blkB1, semA, semB, semw):
    wid = lax.axis_index("s") * 2 + lax.axis_index("c")
    base = wid * BPW

    pltpu.sync_copy(x_hbm.at[pl.ds(base, BPW)], x_v)

    def hash_body(j, _):
        off = j * 16
        xu = x_v[pl.ds(off, 16)].astype(jnp.uint32)
        xh = xu >> jnp.uint32(10)
        xl = xu & jnp.uint32(1023)
        r0 = _hash16(xh, xl, _HA0[0], _HC0[0], B_ROWS)
        r1 = _hash16(xh, xl, _HA0[1], _HC0[1], B_ROWS)
        itile_v[pl.ds(off, 16)] = r0 >> 3
        itile_v[pl.ds(BPW + off, 16)] = r1 >> 3
        iline_v[pl.ds(off, 16)] = r0 & 7
        iline_v[pl.ds(BPW + off, 16)] = r1 & 7
        iw_v[pl.ds(off, 16)] = _hash16(xh, xl, _HA1[0], _HC1[0], W_SIZE)
        iw_v[pl.ds(BPW + off, 16)] = _hash16(xh, xl, _HA1[1], _HC1[1], W_SIZE)
        return 0

    lax.fori_loop(jnp.int32(0), jnp.int32(BPW // 16), hash_body, 0,
                  unroll=False)

    wcps = []
    for ch in range((NH * BPW) // 128):
        wcps.append(pltpu.async_copy(
            w_hbm.at[iw_v.at[pl.ds(ch * 128, 128)]],
            wv_v.at[pl.ds(ch * 128, 128)], semw))
    for cp in wcps:
        cp.wait()

    def fire(w, b0, b1, sem):
        off = w * WIN
        t0 = itile_v[pl.ds(off, 16)]
        t1 = itile_v[pl.ds(BPW + off, 16)]
        for k in range(WIN):
            pltpu.async_copy(table_hbm.at[pl.ds(t0[k], 1)],
                             b0.at[pl.ds(k, 1)], sem)
            pltpu.async_copy(table_hbm.at[pl.ds(t1[k], 1)],
                             b1.at[pl.ds(k, 1)], sem)

    def drain(b0, b1, sem):
        pltpu.make_async_copy(table_hbm.at[pl.ds(0, WIN)], b0, sem).wait()
        pltpu.make_async_copy(table_hbm.at[pl.ds(0, WIN)], b1, sem).wait()

    def combine(w, b0, b1):
        off = w * WIN
        l0 = iline_v[pl.ds(off, 16)]
        l1 = iline_v[pl.ds(BPW + off, 16)]
        w0 = wv_v[pl.ds(off, 16)]
        w1 = wv_v[pl.ds(BPW + off, 16)]
        for k in range(WIN):
            w0s = w0[k]
            w1s = w1[k]
            oa = w0s * b0[k, l0[k], pl.ds(0, 16)] \
                + w1s * b1[k, l1[k], pl.ds(0, 16)]
            ob = w0s * b0[k, l0[k], pl.ds(16, 16)] \
                + w1s * b1[k, l1[k], pl.ds(16, 16)]
            obase = (off + k) * DIM
            out_v[pl.ds(obase, 16)] = oa
            out_v[pl.ds(obase + 16, 16)] = ob

    fire(jnp.int32(0), blkA0, blkA1, semA)

    def pipe_body(s, _):
        wA = s * 2
        wB = wA + 1
        fire(wB, blkB0, blkB1, semB)
        drain(blkA0, blkA1, semA)
        combine(wA, blkA0, blkA1)

        @pl.when(wB + 1 < NWIN)
        def _():
            fire(wB + 1, blkA0, blkA1, semA)

        drain(blkB0, blkB1, semB)
        combine(wB, blkB0, blkB1)
        return 0

    lax.fori_loop(jnp.int32(0), jnp.int32(NWIN // 2), pipe_body, 0,
                  unroll=False)

    pltpu.sync_copy(out_v, out_hbm.at[pl.ds(base * DIM, BPW * DIM)])


@jax.jit
def _run(x_i32, table3, weights):
    mesh = plsc.VectorSubcoreMesh(core_axis_name="c", subcore_axis_name="s")
    f = functools.partial(
        pl.kernel,
        mesh=mesh,
        compiler_params=pltpu.CompilerParams(needs_layout_passes=False),
        out_type=jax.ShapeDtypeStruct((BATCH * DIM,), jnp.float32),
        scratch_types=[
            pltpu.VMEM((BPW,), jnp.int32),
            pltpu.VMEM((NH * BPW,), jnp.int32),
            pltpu.VMEM((NH * BPW,), jnp.int32),
            pltpu.VMEM((NH * BPW,), jnp.int32),
            pltpu.VMEM((NH * BPW,), jnp.float32),
            pltpu.VMEM((BPW * DIM,), jnp.float32),
            pltpu.VMEM((WIN, 8, DIM), jnp.float32),
            pltpu.VMEM((WIN, 8, DIM), jnp.float32),
            pltpu.VMEM((WIN, 8, DIM), jnp.float32),
            pltpu.VMEM((WIN, 8, DIM), jnp.float32),
            pltpu.SemaphoreType.DMA,
            pltpu.SemaphoreType.DMA,
            pltpu.SemaphoreType.DMA,
        ],
    )(_sc_body)
    return f(x_i32, table3, weights)


def kernel(x, table, weights):
    table3 = table.reshape(B_ROWS // 8, 8, DIM)
    out = _run(x.astype(jnp.int32), table3, weights)
    return out.reshape(BATCH, DIM)
